# Initial kernel scaffold; baseline (speedup 1.0000x reference)
import jax
import jax.numpy as jnp
from jax.experimental import pallas as pl


def kernel(pred0, pred1, pred2, boxes, labels):
    raise NotImplementedError("write your pallas kernel here")


# speedup vs baseline: 4.1933x; 4.1933x over previous
import functools

import numpy as np
import jax
import jax.numpy as jnp
from jax import lax
from jax.experimental import pallas as pl
from jax.experimental.pallas import tpu as pltpu
from jax.experimental.pallas import tpu_sc as plsc

_ANCHORS = np.array([[[0.02, 0.02], [0.04, 0.02], [0.02, 0.08]],
                     [[0.06, 0.06], [0.10, 0.04], [0.04, 0.14]],
                     [[0.14, 0.14], [0.22, 0.08], [0.10, 0.24]]], dtype=np.float32)
_NCLS = 5
_B, _A, _N = 32, 3, 20
_HW = (13, 26, 52)
_M = _B * _N
_CHUNKS = (10 * _M) // 128


def _bce0(x):
    return jnp.maximum(x, 0.0) + jnp.log1p(jnp.exp(-jnp.abs(x)))


def _prep_body(boxes_ref, key10_ref, win_ref, tv_ref):
    bx = boxes_ref[...]
    cx, cy, bw, bh = bx[0], bx[1], bx[2], bx[3]
    b_iota = lax.broadcasted_iota(jnp.int32, (_B, _N), 0)
    n1 = lax.broadcasted_iota(jnp.int32, (_N, _N), 0)
    n2 = lax.broadcasted_iota(jnp.int32, (_N, _N), 1)
    later = n2 > n1
    for s in range(3):
        H = W = _HW[s]
        fW = jnp.float32(W)
        gx, gy, gw, gh = cx * fW, cy * fW, bw * fW, bh * fW
        gi = jnp.clip(gx.astype(jnp.int32), 0, W - 1)
        gj = jnp.clip(gy.astype(jnp.int32), 0, H - 1)
        r = []
        for k in range(3):
            awk = np.float32(_ANCHORS[s, k, 0]) * np.float32(W)
            ahk = np.float32(_ANCHORS[s, k, 1]) * np.float32(H)
            inter = jnp.minimum(gw, awk) * jnp.minimum(gh, ahk)
            union = gw * gh + np.float32(awk * ahk) - inter + 1e-6
            r.append(inter / union)
        best = jnp.where(r[1] > r[0], 1, 0)
        best = jnp.where(r[2] > jnp.maximum(r[0], r[1]), 2, best)
        key = ((b_iota * _A + best) * H + gj) * W + gi
        eq = key[:, :, None] == key[:, None, :]
        dup = jnp.any(eq & later[None], axis=2)
        key10_ref[s] = key * 10
        win_ref[s] = jnp.where(dup, 0.0, 1.0)
        tv_ref[s, 0] = gx - gi.astype(jnp.float32)
        tv_ref[s, 1] = gy - gj.astype(jnp.float32)
        tv_ref[s, 2] = gw
        tv_ref[s, 3] = gh


def _prep(boxes_t):
    return pl.pallas_call(
        _prep_body,
        out_shape=[
            jax.ShapeDtypeStruct((3, _B, _N), jnp.int32),
            jax.ShapeDtypeStruct((3, _B, _N), jnp.float32),
            jax.ShapeDtypeStruct((3, 4, _B, _N), jnp.float32),
        ],
    )(boxes_t)


def _gather_body(f0, f1, f2, eidx, out, idx_v, row_v, sem):
    wid = lax.axis_index("s") * 2 + lax.axis_index("c")
    tabs = [f0, f1, f2]
    for j in range(2):
        cid = wid + 32 * j

        @pl.when(cid < _CHUNKS)
        def _():
            for s in range(3):
                pltpu.sync_copy(eidx.at[s, cid], idx_v)
                pltpu.async_copy(tabs[s].at[idx_v], row_v, sem).wait()
                pltpu.sync_copy(row_v, out.at[s, cid])


def _gather(f0, f1, f2, eidx):
    gk = pl.kernel(
        _gather_body,
        out_type=jax.ShapeDtypeStruct((3, _CHUNKS, 128), jnp.float32),
        mesh=plsc.VectorSubcoreMesh(core_axis_name="c", subcore_axis_name="s"),
        scratch_types=[
            pltpu.VMEM((128,), jnp.int32),
            pltpu.VMEM((128,), jnp.float32),
            pltpu.SemaphoreType.DMA,
        ],
    )
    return gk(f0, f1, f2, eidx)


_ROWS_PB = 8


def _dense_body(p0_ref, p1_ref, p2_ref, out_ref):
    @pl.when(pl.program_id(0) == 0)
    def _():
        for s in range(3):
            out_ref[s] = 0.0

    for s, ref in enumerate([p0_ref, p1_ref, p2_ref]):
        x = ref[...]
        ch = lax.broadcasted_iota(jnp.int32, x.shape, 1) % 10
        out_ref[s] += jnp.sum(jnp.where(ch == 4, _bce0(x), 0.0))


def _dense(p0r, p1r, p2r):
    nsteps = (_B * _A) // _ROWS_PB
    specs = [
        pl.BlockSpec((_ROWS_PB, p.shape[1]), lambda i: (i, 0))
        for p in (p0r, p1r, p2r)
    ]
    return pl.pallas_call(
        _dense_body,
        grid=(nsteps,),
        in_specs=specs,
        out_specs=pl.BlockSpec(memory_space=pltpu.SMEM),
        out_shape=jax.ShapeDtypeStruct((3,), jnp.float32),
    )(p0r, p1r, p2r)


def _combine_body(gath_ref, tv_ref, win_ref, lab_ref, ds_ref, out_ref):
    total = jnp.float32(0.0)
    lab = lab_ref[...]
    ci = lax.broadcasted_iota(jnp.int32, (_NCLS, _M), 0)
    for s in range(3):
        H = W = _HW[s]
        g = gath_ref[s]
        w = win_ref[s]
        x4 = g[4:5, :]
        s0 = _bce0(x4)
        n_obj = jnp.sum(w)
        s0m = jnp.sum(w * s0)
        sxm = jnp.sum(w * x4)
        sxy = jnp.sum(w * ((g[0:1] - tv_ref[s, 0]) ** 2 + (g[1:2] - tv_ref[s, 1]) ** 2))
        swh = jnp.sum(w * ((jnp.abs(g[2:3]) - tv_ref[s, 2]) ** 2
                           + (jnp.abs(g[3:4]) - tv_ref[s, 3]) ** 2))
        xc = g[5:10, :]
        onehot = ci == lab
        scls = jnp.sum(w * (_bce0(xc) - jnp.where(onehot, xc, 0.0)))
        n_noobj = jnp.float32(_B * _A * H * W) - n_obj
        total = total + (s0m - sxm) / n_obj \
            + 0.5 * (ds_ref[s] - s0m) / n_noobj \
            + 5.0 * (sxy + swh) / (2.0 * n_obj) \
            + scls / (jnp.float32(_NCLS) * n_obj)
    out_ref[0] = total


def _combine(gath, tv, win, lab, dsums):
    return pl.pallas_call(
        _combine_body,
        in_specs=[
            pl.BlockSpec(),
            pl.BlockSpec(),
            pl.BlockSpec(),
            pl.BlockSpec(),
            pl.BlockSpec(memory_space=pltpu.SMEM),
        ],
        out_specs=pl.BlockSpec(memory_space=pltpu.SMEM),
        out_shape=jax.ShapeDtypeStruct((1,), jnp.float32),
    )(gath, tv, win, lab, dsums)


def kernel(pred0, pred1, pred2, boxes, labels):
    boxes_t = jnp.transpose(boxes, (2, 0, 1))
    key10, win, tv = _prep(boxes_t)
    eidx = (key10.reshape(3, 1, _M)
            + jnp.arange(10, dtype=jnp.int32).reshape(1, 10, 1)).reshape(3, _CHUNKS, 128)
    gath = _gather(pred0.reshape(-1), pred1.reshape(-1), pred2.reshape(-1), eidx)
    dsums = _dense(pred0.reshape(_B * _A, -1),
                   pred1.reshape(_B * _A, -1),
                   pred2.reshape(_B * _A, -1))
    tot = _combine(gath.reshape(3, 10, _M),
                   tv.reshape(3, 4, 1, _M),
                   win.reshape(3, 1, _M),
                   labels.reshape(1, _M).astype(jnp.int32),
                   dsums)
    return tot[0]


# speedup vs baseline: 17.6945x; 4.2198x over previous
import functools

import numpy as np
import jax
import jax.numpy as jnp
from jax import lax
from jax.experimental import pallas as pl
from jax.experimental.pallas import tpu as pltpu
from jax.experimental.pallas import tpu_sc as plsc

_ANCHORS = np.array([[[0.02, 0.02], [0.04, 0.02], [0.02, 0.08]],
                     [[0.06, 0.06], [0.10, 0.04], [0.04, 0.14]],
                     [[0.14, 0.14], [0.22, 0.08], [0.10, 0.24]]], dtype=np.float32)
_NCLS = 5
_B, _A, _N = 32, 3, 20
_HW = (13, 26, 52)
_M = _B * _N
_CHUNKS = (10 * _M) // 128


def _bce0(x):
    return jnp.maximum(x, 0.0) + jnp.log1p(jnp.exp(-jnp.abs(x)))


def _prep_body(boxes_ref, base_ref, win_ref, tv_ref):
    bx = boxes_ref[...]
    cx, cy, bw, bh = bx[0], bx[1], bx[2], bx[3]
    b_iota = lax.broadcasted_iota(jnp.int32, (_B, _N), 0)
    n1 = lax.broadcasted_iota(jnp.int32, (_N, _N), 0)
    n2 = lax.broadcasted_iota(jnp.int32, (_N, _N), 1)
    later = n2 > n1
    for s in range(3):
        H = W = _HW[s]
        fW = jnp.float32(W)
        gx, gy, gw, gh = cx * fW, cy * fW, bw * fW, bh * fW
        gi = jnp.clip(gx.astype(jnp.int32), 0, W - 1)
        gj = jnp.clip(gy.astype(jnp.int32), 0, H - 1)
        r = []
        for k in range(3):
            awk = np.float32(_ANCHORS[s, k, 0]) * np.float32(W)
            ahk = np.float32(_ANCHORS[s, k, 1]) * np.float32(H)
            inter = jnp.minimum(gw, awk) * jnp.minimum(gh, ahk)
            union = gw * gh + np.float32(awk * ahk) - inter + 1e-6
            r.append(inter / union)
        best = jnp.where(r[1] > r[0], 1, 0)
        best = jnp.where(r[2] > jnp.maximum(r[0], r[1]), 2, best)
        ahj = best * H + gj
        if s == 0:
            base = ahj * (10 * 13 * 32) + gi * 32 + b_iota
        else:
            base = ahj * (10 * _B * W) + b_iota * W + gi
        eq = base[:, :, None] == base[:, None, :]
        dup = jnp.any(eq & later[None], axis=2)
        base_ref[s] = base
        win_ref[s] = jnp.where(dup, 0.0, 1.0)
        tv_ref[s, 0] = gx - gi.astype(jnp.float32)
        tv_ref[s, 1] = gy - gj.astype(jnp.float32)
        tv_ref[s, 2] = gw
        tv_ref[s, 3] = gh


def _prep(boxes_t):
    return pl.pallas_call(
        _prep_body,
        out_shape=[
            jax.ShapeDtypeStruct((3, _B, _N), jnp.int32),
            jax.ShapeDtypeStruct((3, _B, _N), jnp.float32),
            jax.ShapeDtypeStruct((3, 4, _B, _N), jnp.float32),
        ],
    )(boxes_t)


def _gather_body(f0, f1, f2, eidx, out, idx_v, row_v, sem):
    wid = lax.axis_index("s") * 2 + lax.axis_index("c")
    tabs = [f0, f1, f2]
    for j in range(2):
        cid = wid + 32 * j

        @pl.when(cid < _CHUNKS)
        def _():
            for s in range(3):
                pltpu.sync_copy(eidx.at[s, cid], idx_v)
                pltpu.async_copy(tabs[s].at[idx_v], row_v, sem).wait()
                pltpu.sync_copy(row_v, out.at[s, cid])


def _gather(f0, f1, f2, eidx):
    gk = pl.kernel(
        _gather_body,
        out_type=jax.ShapeDtypeStruct((3, _CHUNKS, 128), jnp.float32),
        mesh=plsc.VectorSubcoreMesh(core_axis_name="c", subcore_axis_name="s"),
        scratch_types=[
            pltpu.VMEM((128,), jnp.int32),
            pltpu.VMEM((128,), jnp.float32),
            pltpu.SemaphoreType.DMA,
        ],
    )
    return gk(f0, f1, f2, eidx)


def _dense_body(p0_ref, p1_ref, p2_ref, out_ref):
    @pl.when(pl.program_id(0) == 0)
    def _():
        for s in range(3):
            out_ref[s] = 0.0

    for s, ref in enumerate([p0_ref, p1_ref, p2_ref]):
        out_ref[s] += jnp.sum(_bce0(ref[...]))


def _dense(v0, v1, v2):
    specs = [
        pl.BlockSpec((1, 13, 1, 13, 32), lambda a: (a, 0, 4, 0, 0)),
        pl.BlockSpec((1, 26, 1, 32, 26), lambda a: (a, 0, 4, 0, 0)),
        pl.BlockSpec((1, 52, 1, 32, 52), lambda a: (a, 0, 4, 0, 0)),
    ]
    return pl.pallas_call(
        _dense_body,
        grid=(3,),
        in_specs=specs,
        out_specs=pl.BlockSpec(memory_space=pltpu.SMEM),
        out_shape=jax.ShapeDtypeStruct((3,), jnp.float32),
    )(v0, v1, v2)


def _combine_body(gath_ref, tv_ref, win_ref, lab_ref, ds_ref, out_ref):
    total = jnp.float32(0.0)
    lab = lab_ref[...]
    ci = lax.broadcasted_iota(jnp.int32, (_NCLS, _M), 0)
    for s in range(3):
        H = W = _HW[s]
        g = gath_ref[s]
        w = win_ref[s]
        x4 = g[4:5, :]
        s0 = _bce0(x4)
        n_obj = jnp.sum(w)
        s0m = jnp.sum(w * s0)
        sxm = jnp.sum(w * x4)
        sxy = jnp.sum(w * ((g[0:1] - tv_ref[s, 0]) ** 2 + (g[1:2] - tv_ref[s, 1]) ** 2))
        swh = jnp.sum(w * ((jnp.abs(g[2:3]) - tv_ref[s, 2]) ** 2
                           + (jnp.abs(g[3:4]) - tv_ref[s, 3]) ** 2))
        xc = g[5:10, :]
        onehot = ci == lab
        scls = jnp.sum(w * (_bce0(xc) - jnp.where(onehot, xc, 0.0)))
        n_noobj = jnp.float32(_B * _A * H * W) - n_obj
        total = total + (s0m - sxm) / n_obj \
            + 0.5 * (ds_ref[s] - s0m) / n_noobj \
            + 5.0 * (sxy + swh) / (2.0 * n_obj) \
            + scls / (jnp.float32(_NCLS) * n_obj)
    out_ref[0] = total


def _combine(gath, tv, win, lab, dsums):
    return pl.pallas_call(
        _combine_body,
        in_specs=[
            pl.BlockSpec(),
            pl.BlockSpec(),
            pl.BlockSpec(),
            pl.BlockSpec(),
            pl.BlockSpec(memory_space=pltpu.SMEM),
        ],
        out_specs=pl.BlockSpec(memory_space=pltpu.SMEM),
        out_shape=jax.ShapeDtypeStruct((1,), jnp.float32),
    )(gath, tv, win, lab, dsums)


def kernel(pred0, pred1, pred2, boxes, labels):
    v0 = jnp.transpose(pred0, (1, 2, 4, 3, 0))
    v1 = jnp.transpose(pred1, (1, 2, 4, 0, 3))
    v2 = jnp.transpose(pred2, (1, 2, 4, 0, 3))
    boxes_t = jnp.transpose(boxes, (2, 0, 1))
    base, win, tv = _prep(boxes_t)
    strides = (13 * 32, _B * 26, _B * 52)
    eidx = jnp.stack([
        base[s].reshape(1, _M)
        + (jnp.arange(10, dtype=jnp.int32) * strides[s]).reshape(10, 1)
        for s in range(3)
    ]).reshape(3, _CHUNKS, 128)
    gath = _gather(v0.reshape(-1), v1.reshape(-1), v2.reshape(-1), eidx)
    dsums = _dense(v0, v1, v2)
    tot = _combine(gath.reshape(3, 10, _M),
                   tv.reshape(3, 4, 1, _M),
                   win.reshape(3, 1, _M),
                   labels.reshape(1, _M).astype(jnp.int32),
                   dsums)
    return tot[0]


# speedup vs baseline: 17.7369x; 1.0024x over previous
import functools

import numpy as np
import jax
import jax.numpy as jnp
from jax import lax
from jax.experimental import pallas as pl
from jax.experimental.pallas import tpu as pltpu
from jax.experimental.pallas import tpu_sc as plsc

_ANCHORS = np.array([[[0.02, 0.02], [0.04, 0.02], [0.02, 0.08]],
                     [[0.06, 0.06], [0.10, 0.04], [0.04, 0.14]],
                     [[0.14, 0.14], [0.22, 0.08], [0.10, 0.24]]], dtype=np.float32)
_NCLS = 5
_B, _A, _N = 32, 3, 20
_HW = (13, 26, 52)
_M = _B * _N
_CHUNKS = (10 * _M) // 128


def _bce0(x):
    return jnp.maximum(x, 0.0) + jnp.log1p(jnp.exp(-jnp.abs(x)))


def _prep_body(boxes_ref, base_ref, win_ref, tv_ref):
    bx = boxes_ref[...]
    cx, cy, bw, bh = bx[0], bx[1], bx[2], bx[3]
    b_iota = lax.broadcasted_iota(jnp.int32, (_B, _N), 0)
    n1 = lax.broadcasted_iota(jnp.int32, (_N, _N), 0)
    n2 = lax.broadcasted_iota(jnp.int32, (_N, _N), 1)
    later = n2 > n1
    for s in range(3):
        H = W = _HW[s]
        fW = jnp.float32(W)
        gx, gy, gw, gh = cx * fW, cy * fW, bw * fW, bh * fW
        gi = jnp.clip(gx.astype(jnp.int32), 0, W - 1)
        gj = jnp.clip(gy.astype(jnp.int32), 0, H - 1)
        r = []
        for k in range(3):
            awk = np.float32(_ANCHORS[s, k, 0]) * np.float32(W)
            ahk = np.float32(_ANCHORS[s, k, 1]) * np.float32(H)
            inter = jnp.minimum(gw, awk) * jnp.minimum(gh, ahk)
            union = gw * gh + np.float32(awk * ahk) - inter + 1e-6
            r.append(inter / union)
        best = jnp.where(r[1] > r[0], 1, 0)
        best = jnp.where(r[2] > jnp.maximum(r[0], r[1]), 2, best)
        ahj = best * H + gj
        if s == 0:
            base = ahj * (10 * 13 * 32) + gi * 32 + b_iota
        else:
            base = ahj * (10 * _B * W) + b_iota * W + gi
        eq = base[:, :, None] == base[:, None, :]
        dup = jnp.any(eq & later[None], axis=2)
        base_ref[s] = base
        win_ref[s] = jnp.where(dup, 0.0, 1.0)
        tv_ref[s, 0] = gx - gi.astype(jnp.float32)
        tv_ref[s, 1] = gy - gj.astype(jnp.float32)
        tv_ref[s, 2] = gw
        tv_ref[s, 3] = gh


def _prep(boxes_t):
    return pl.pallas_call(
        _prep_body,
        out_shape=[
            jax.ShapeDtypeStruct((3, _B, _N), jnp.int32),
            jax.ShapeDtypeStruct((3, _B, _N), jnp.float32),
            jax.ShapeDtypeStruct((3, 4, _B, _N), jnp.float32),
        ],
    )(boxes_t)


def _gather01_body(f0, f1, eidx, out, idx_v, row_v, sem):
    wid = lax.axis_index("s") * 2 + lax.axis_index("c")
    tabs = [f0, f1]
    for j in range(2):
        cid = wid + 32 * j

        @pl.when(cid < _CHUNKS)
        def _():
            for s in range(2):
                pltpu.sync_copy(eidx.at[s, cid], idx_v)
                pltpu.async_copy(tabs[s].at[idx_v], row_v, sem).wait()
                pltpu.sync_copy(row_v, out.at[s, cid])


def _gather2_body(f2, eidx, out, idx_v, row_v, sem):
    wid = lax.axis_index("s") * 2 + lax.axis_index("c")
    for j in range(2):
        cid = wid + 32 * j

        @pl.when(cid < _CHUNKS)
        def _():
            pltpu.sync_copy(eidx.at[cid], idx_v)
            pltpu.async_copy(f2.at[idx_v], row_v, sem).wait()
            pltpu.sync_copy(row_v, out.at[cid])


_SC_SCRATCH = [
    pltpu.VMEM((128,), jnp.int32),
    pltpu.VMEM((128,), jnp.float32),
    pltpu.SemaphoreType.DMA,
]


def _gather01(f0, f1, eidx):
    gk = pl.kernel(
        _gather01_body,
        out_type=jax.ShapeDtypeStruct((2, _CHUNKS, 128), jnp.float32),
        mesh=plsc.VectorSubcoreMesh(core_axis_name="c", subcore_axis_name="s"),
        scratch_types=_SC_SCRATCH,
    )
    return gk(f0, f1, eidx)


def _gather2(f2, eidx):
    gk = pl.kernel(
        _gather2_body,
        out_type=jax.ShapeDtypeStruct((_CHUNKS, 128), jnp.float32),
        mesh=plsc.VectorSubcoreMesh(core_axis_name="c", subcore_axis_name="s"),
        scratch_types=_SC_SCRATCH,
    )
    return gk(f2, eidx)


def _dense_body(p0_ref, p1_ref, p2_ref, out_ref):
    @pl.when(pl.program_id(0) == 0)
    def _():
        for s in range(3):
            out_ref[s] = 0.0

    for s, ref in enumerate([p0_ref, p1_ref, p2_ref]):
        out_ref[s] += jnp.sum(_bce0(ref[...]))


def _dense(v0, v1, v2):
    specs = [
        pl.BlockSpec((1, 13, 1, 13, 32), lambda a: (a, 0, 4, 0, 0)),
        pl.BlockSpec((1, 26, 1, 32, 26), lambda a: (a, 0, 4, 0, 0)),
        pl.BlockSpec((1, 52, 1, 32, 52), lambda a: (a, 0, 4, 0, 0)),
    ]
    return pl.pallas_call(
        _dense_body,
        grid=(3,),
        in_specs=specs,
        out_specs=pl.BlockSpec(memory_space=pltpu.SMEM),
        out_shape=jax.ShapeDtypeStruct((3,), jnp.float32),
    )(v0, v1, v2)


def _combine_body(gath_ref, tv_ref, win_ref, lab_ref, ds_ref, out_ref):
    total = jnp.float32(0.0)
    lab = lab_ref[...]
    ci = lax.broadcasted_iota(jnp.int32, (_NCLS, _M), 0)
    for s in range(3):
        H = W = _HW[s]
        g = gath_ref[s]
        w = win_ref[s]
        x4 = g[4:5, :]
        s0 = _bce0(x4)
        n_obj = jnp.sum(w)
        s0m = jnp.sum(w * s0)
        sxm = jnp.sum(w * x4)
        sxy = jnp.sum(w * ((g[0:1] - tv_ref[s, 0]) ** 2 + (g[1:2] - tv_ref[s, 1]) ** 2))
        swh = jnp.sum(w * ((jnp.abs(g[2:3]) - tv_ref[s, 2]) ** 2
                           + (jnp.abs(g[3:4]) - tv_ref[s, 3]) ** 2))
        xc = g[5:10, :]
        onehot = ci == lab
        scls = jnp.sum(w * (_bce0(xc) - jnp.where(onehot, xc, 0.0)))
        n_noobj = jnp.float32(_B * _A * H * W) - n_obj
        total = total + (s0m - sxm) / n_obj \
            + 0.5 * (ds_ref[s] - s0m) / n_noobj \
            + 5.0 * (sxy + swh) / (2.0 * n_obj) \
            + scls / (jnp.float32(_NCLS) * n_obj)
    out_ref[0] = total


def _combine(gath, tv, win, lab, dsums):
    return pl.pallas_call(
        _combine_body,
        in_specs=[
            pl.BlockSpec(),
            pl.BlockSpec(),
            pl.BlockSpec(),
            pl.BlockSpec(),
            pl.BlockSpec(memory_space=pltpu.SMEM),
        ],
        out_specs=pl.BlockSpec(memory_space=pltpu.SMEM),
        out_shape=jax.ShapeDtypeStruct((1,), jnp.float32),
    )(gath, tv, win, lab, dsums)


def kernel(pred0, pred1, pred2, boxes, labels):
    v0 = jnp.transpose(pred0, (1, 2, 4, 3, 0))
    v1 = jnp.transpose(pred1, (1, 2, 4, 0, 3))
    v2 = jnp.transpose(pred2, (1, 2, 4, 0, 3))
    boxes_t = jnp.transpose(boxes, (2, 0, 1))
    base, win, tv = _prep(boxes_t)
    strides = (13 * 32, _B * 26, _B * 52)
    eidx = jnp.stack([
        base[s].reshape(1, _M)
        + (jnp.arange(10, dtype=jnp.int32) * strides[s]).reshape(10, 1)
        for s in range(3)
    ]).reshape(3, _CHUNKS, 128)
    gath01 = _gather01(v0.reshape(-1), v1.reshape(-1), eidx[:2])
    gath2 = _gather2(v2.reshape(-1), eidx[2])
    gath = jnp.concatenate([gath01, gath2.reshape(1, _CHUNKS, 128)])
    dsums = _dense(v0, v1, v2)
    tot = _combine(gath.reshape(3, 10, _M),
                   tv.reshape(3, 4, 1, _M),
                   win.reshape(3, 1, _M),
                   labels.reshape(1, _M).astype(jnp.int32),
                   dsums)
    return tot[0]


# speedup vs baseline: 17.7417x; 1.0003x over previous
import numpy as np
import jax
import jax.numpy as jnp
from jax import lax
from jax.experimental import pallas as pl
from jax.experimental.pallas import tpu as pltpu
from jax.experimental.pallas import tpu_sc as plsc

_ANCHORS = np.array([[[0.02, 0.02], [0.04, 0.02], [0.02, 0.08]],
                     [[0.06, 0.06], [0.10, 0.04], [0.04, 0.14]],
                     [[0.14, 0.14], [0.22, 0.08], [0.10, 0.24]]], dtype=np.float32)
_NCLS = 5
_B, _A, _N = 32, 3, 20
_HW = (13, 26, 52)
_M = _B * _N
_CHUNKS = (10 * _M) // 128


def _bce0(x):
    return jnp.maximum(x, 0.0) + jnp.log1p(jnp.exp(-jnp.abs(x)))


def _prep_body(boxes_ref, base_ref, win_ref, tv_ref):
    bx = boxes_ref[...]
    cx, cy, bw, bh = bx[0], bx[1], bx[2], bx[3]
    b_iota = lax.broadcasted_iota(jnp.int32, (_B, _N), 0)
    n1 = lax.broadcasted_iota(jnp.int32, (_N, _N), 0)
    n2 = lax.broadcasted_iota(jnp.int32, (_N, _N), 1)
    later = n2 > n1
    for s in range(3):
        H = W = _HW[s]
        fW = jnp.float32(W)
        gx, gy, gw, gh = cx * fW, cy * fW, bw * fW, bh * fW
        gi = jnp.clip(gx.astype(jnp.int32), 0, W - 1)
        gj = jnp.clip(gy.astype(jnp.int32), 0, H - 1)
        r = []
        for k in range(3):
            awk = np.float32(_ANCHORS[s, k, 0]) * np.float32(W)
            ahk = np.float32(_ANCHORS[s, k, 1]) * np.float32(H)
            inter = jnp.minimum(gw, awk) * jnp.minimum(gh, ahk)
            union = gw * gh + np.float32(awk * ahk) - inter + 1e-6
            r.append(inter / union)
        best = jnp.where(r[1] > r[0], 1, 0)
        best = jnp.where(r[2] > jnp.maximum(r[0], r[1]), 2, best)
        ahj = best * H + gj
        if s == 0:
            base = ahj * (10 * 13 * 32) + gi * 32 + b_iota
        else:
            base = ahj * (10 * _B * W) + b_iota * W + gi
        eq = base[:, :, None] == base[:, None, :]
        dup = jnp.any(eq & later[None], axis=2)
        base_ref[s] = base
        win_ref[s] = jnp.where(dup, 0.0, 1.0)
        tv_ref[s, 0] = gx - gi.astype(jnp.float32)
        tv_ref[s, 1] = gy - gj.astype(jnp.float32)
        tv_ref[s, 2] = gw
        tv_ref[s, 3] = gh


def _prep(boxes_t):
    return pl.pallas_call(
        _prep_body,
        out_shape=[
            jax.ShapeDtypeStruct((3, _B, _N), jnp.int32),
            jax.ShapeDtypeStruct((3, _B, _N), jnp.float32),
            jax.ShapeDtypeStruct((3, 4, _B, _N), jnp.float32),
        ],
    )(boxes_t)


def _gather01_body(f0, f1, eidx, out, idx_v, row_v, sem):
    wid = lax.axis_index("s") * 2 + lax.axis_index("c")
    tabs = [f0, f1]
    for j in range(2):
        cid = wid + 32 * j

        @pl.when(cid < _CHUNKS)
        def _():
            for s in range(2):
                pltpu.sync_copy(eidx.at[s, cid], idx_v)
                pltpu.async_copy(tabs[s].at[idx_v], row_v, sem).wait()
                pltpu.sync_copy(row_v, out.at[s, cid])


def _gather2_body(f2, eidx, out, idx_v, row_v, sem):
    wid = lax.axis_index("s") * 2 + lax.axis_index("c")
    for j in range(2):
        cid = wid + 32 * j

        @pl.when(cid < _CHUNKS)
        def _():
            pltpu.sync_copy(eidx.at[cid], idx_v)
            pltpu.async_copy(f2.at[idx_v], row_v, sem).wait()
            pltpu.sync_copy(row_v, out.at[cid])


_SC_SCRATCH = [
    pltpu.VMEM((128,), jnp.int32),
    pltpu.VMEM((128,), jnp.float32),
    pltpu.SemaphoreType.DMA,
]


def _gather01(f0, f1, eidx):
    gk = pl.kernel(
        _gather01_body,
        out_type=jax.ShapeDtypeStruct((2, _CHUNKS, 128), jnp.float32),
        mesh=plsc.VectorSubcoreMesh(core_axis_name="c", subcore_axis_name="s"),
        scratch_types=_SC_SCRATCH,
    )
    return gk(f0, f1, eidx)


def _gather2(f2, eidx):
    gk = pl.kernel(
        _gather2_body,
        out_type=jax.ShapeDtypeStruct((_CHUNKS, 128), jnp.float32),
        mesh=plsc.VectorSubcoreMesh(core_axis_name="c", subcore_axis_name="s"),
        scratch_types=_SC_SCRATCH,
    )
    return gk(f2, eidx)


def _dense_body(p0_ref, p1_ref, p2_ref, out_ref):
    @pl.when(pl.program_id(0) == 0)
    def _():
        for s in range(3):
            out_ref[s] = 0.0

    for s, ref in enumerate([p0_ref, p1_ref, p2_ref]):
        out_ref[s] += jnp.sum(_bce0(ref[...]))


def _dense(v0, v1, v2):
    specs = [
        pl.BlockSpec((1, 13, 1, 13, 32), lambda a: (a, 0, 4, 0, 0)),
        pl.BlockSpec((1, 26, 1, 32, 26), lambda a: (a, 0, 4, 0, 0)),
        pl.BlockSpec((1, 52, 1, 32, 52), lambda a: (a, 0, 4, 0, 0)),
    ]
    return pl.pallas_call(
        _dense_body,
        grid=(3,),
        in_specs=specs,
        out_specs=pl.BlockSpec(memory_space=pltpu.SMEM),
        out_shape=jax.ShapeDtypeStruct((3,), jnp.float32),
    )(v0, v1, v2)


def _combine_body(gath_ref, tv_ref, win_ref, lab_ref, ds_ref, out_ref):
    total = jnp.float32(0.0)
    lab = lab_ref[...]
    ci = lax.broadcasted_iota(jnp.int32, (_NCLS, _M), 0)
    for s in range(3):
        H = W = _HW[s]
        g = gath_ref[s]
        w = win_ref[s]
        x4 = g[4:5, :]
        s0 = _bce0(x4)
        n_obj = jnp.sum(w)
        s0m = jnp.sum(w * s0)
        sxm = jnp.sum(w * x4)
        sxy = jnp.sum(w * ((g[0:1] - tv_ref[s, 0]) ** 2 + (g[1:2] - tv_ref[s, 1]) ** 2))
        swh = jnp.sum(w * ((jnp.abs(g[2:3]) - tv_ref[s, 2]) ** 2
                           + (jnp.abs(g[3:4]) - tv_ref[s, 3]) ** 2))
        xc = g[5:10, :]
        onehot = ci == lab
        scls = jnp.sum(w * (_bce0(xc) - jnp.where(onehot, xc, 0.0)))
        n_noobj = jnp.float32(_B * _A * H * W) - n_obj
        total = total + (s0m - sxm) / n_obj \
            + 0.5 * (ds_ref[s] - s0m) / n_noobj \
            + 5.0 * (sxy + swh) / (2.0 * n_obj) \
            + scls / (jnp.float32(_NCLS) * n_obj)
    out_ref[0] = total


def _combine(gath, tv, win, lab, dsums):
    return pl.pallas_call(
        _combine_body,
        in_specs=[
            pl.BlockSpec(),
            pl.BlockSpec(),
            pl.BlockSpec(),
            pl.BlockSpec(),
            pl.BlockSpec(memory_space=pltpu.SMEM),
        ],
        out_specs=pl.BlockSpec(memory_space=pltpu.SMEM),
        out_shape=jax.ShapeDtypeStruct((1,), jnp.float32),
    )(gath, tv, win, lab, dsums)


def kernel(pred0, pred1, pred2, boxes, labels):
    v0 = jnp.transpose(pred0, (1, 2, 4, 3, 0))
    v1 = jnp.transpose(pred1, (1, 2, 4, 0, 3))
    v2 = jnp.transpose(pred2, (1, 2, 4, 0, 3))
    boxes_t = jnp.transpose(boxes, (2, 0, 1))
    base, win, tv = _prep(boxes_t)
    strides = (13 * 32, _B * 26, _B * 52)
    eidx = jnp.stack([
        base[s].reshape(1, _M)
        + (jnp.arange(10, dtype=jnp.int32) * strides[s]).reshape(10, 1)
        for s in range(3)
    ]).reshape(3, _CHUNKS, 128)
    gath01 = _gather01(v0.reshape(-1), v1.reshape(-1), eidx[:2])
    gath2 = _gather2(v2.reshape(-1), eidx[2])
    gath = jnp.concatenate([gath01, gath2.reshape(1, _CHUNKS, 128)])
    dsums = _dense(v0, v1, v2)
    tot = _combine(gath.reshape(3, 10, _M),
                   tv.reshape(3, 4, 1, _M),
                   win.reshape(3, 1, _M),
                   labels.reshape(1, _M).astype(jnp.int32),
                   dsums)
    return tot[0]


# speedup vs baseline: 24.4051x; 1.3756x over previous
import numpy as np
import jax
import jax.numpy as jnp
from jax import lax
from jax.experimental import pallas as pl
from jax.experimental.pallas import tpu as pltpu
from jax.experimental.pallas import tpu_sc as plsc

_ANCHORS = np.array([[[0.02, 0.02], [0.04, 0.02], [0.02, 0.08]],
                     [[0.06, 0.06], [0.10, 0.04], [0.04, 0.14]],
                     [[0.14, 0.14], [0.22, 0.08], [0.10, 0.24]]], dtype=np.float32)
_NCLS = 5
_B, _A, _N = 32, 3, 20
_HW = (13, 26, 52)
_M = _B * _N
_CHUNKS = (10 * _M) // 128


def _bce0(x):
    return jnp.maximum(x, 0.0) + jnp.log1p(jnp.exp(-jnp.abs(x)))


def _prep_body(boxes_ref, base_ref, lane_ref, win_ref, tv_ref):
    bx = boxes_ref[...]
    cx, cy, bw, bh = bx[0], bx[1], bx[2], bx[3]
    b_iota = lax.broadcasted_iota(jnp.int32, (_B, _N), 0)
    n1 = lax.broadcasted_iota(jnp.int32, (_N, _N), 0)
    n2 = lax.broadcasted_iota(jnp.int32, (_N, _N), 1)
    later = n2 > n1
    for s in range(3):
        H = W = _HW[s]
        fW = jnp.float32(W)
        gx, gy, gw, gh = cx * fW, cy * fW, bw * fW, bh * fW
        gi = jnp.clip(gx.astype(jnp.int32), 0, W - 1)
        gj = jnp.clip(gy.astype(jnp.int32), 0, H - 1)
        r = []
        for k in range(3):
            awk = np.float32(_ANCHORS[s, k, 0]) * np.float32(W)
            ahk = np.float32(_ANCHORS[s, k, 1]) * np.float32(H)
            inter = jnp.minimum(gw, awk) * jnp.minimum(gh, ahk)
            union = gw * gh + np.float32(awk * ahk) - inter + 1e-6
            r.append(inter / union)
        best = jnp.where(r[1] > r[0], 1, 0)
        best = jnp.where(r[2] > jnp.maximum(r[0], r[1]), 2, best)
        ahj = best * H + gj
        if s == 0:
            base = ahj * (10 * 13 * 32) + gi * 32 + b_iota
            ln = jnp.zeros_like(base)
        elif s == 1:
            base = ahj * (10 * _B * W) + b_iota * W + gi
            ln = jnp.zeros_like(base)
        else:
            base = ahj * (10 * _B) + b_iota
            ln = gi
        key = base * 64 + ln
        eq = key[:, :, None] == key[:, None, :]
        dup = jnp.any(eq & later[None], axis=2)
        base_ref[s] = base
        lane_ref[s] = ln
        win_ref[s] = jnp.where(dup, 0.0, 1.0)
        tv_ref[s, 0] = gx - gi.astype(jnp.float32)
        tv_ref[s, 1] = gy - gj.astype(jnp.float32)
        tv_ref[s, 2] = gw
        tv_ref[s, 3] = gh


def _prep(boxes_t):
    return pl.pallas_call(
        _prep_body,
        out_shape=[
            jax.ShapeDtypeStruct((3, _B, _N), jnp.int32),
            jax.ShapeDtypeStruct((3, _B, _N), jnp.int32),
            jax.ShapeDtypeStruct((3, _B, _N), jnp.float32),
            jax.ShapeDtypeStruct((3, 4, _B, _N), jnp.float32),
        ],
    )(boxes_t)


def _gather01_body(f0, f1, eidx, out, idx_v, row_v, sem):
    wid = lax.axis_index("s") * 2 + lax.axis_index("c")
    tabs = [f0, f1]
    for j in range(2):
        cid = wid + 32 * j

        @pl.when(cid < _CHUNKS)
        def _():
            for s in range(2):
                pltpu.sync_copy(eidx.at[s, cid], idx_v)
                pltpu.async_copy(tabs[s].at[idx_v], row_v, sem).wait()
                pltpu.sync_copy(row_v, out.at[s, cid])


_E2 = 200
_E2P = 208


def _gather2_body(y2, ridx, lidx, out, ridx_v, lidx_v, rows_v, out_v, sem):
    wid = lax.axis_index("s") * 2 + lax.axis_index("c")
    pltpu.sync_copy(ridx.at[wid], ridx_v)
    pltpu.sync_copy(lidx.at[wid], lidx_v)
    cps = []
    for k in range(_E2P // 16):
        rv = ridx_v[pl.ds(16 * k, 16)]
        for t in range(16):
            i = 16 * k + t
            cps.append(pltpu.async_copy(y2.at[rv[t]], rows_v.at[i], sem))
    for cp in cps:
        cp.wait()
    for i in range(_E2P // 16):
        rr = lax.iota(jnp.int32, 16) + 16 * i
        ll = lidx_v[pl.ds(16 * i, 16)]
        out_v[pl.ds(16 * i, 16)] = plsc.load_gather(rows_v, [rr, ll])
    pltpu.sync_copy(out_v, out.at[wid])


_SC_SCRATCH = [
    pltpu.VMEM((128,), jnp.int32),
    pltpu.VMEM((128,), jnp.float32),
    pltpu.SemaphoreType.DMA,
]


def _gather01(f0, f1, eidx):
    gk = pl.kernel(
        _gather01_body,
        out_type=jax.ShapeDtypeStruct((2, _CHUNKS, 128), jnp.float32),
        mesh=plsc.VectorSubcoreMesh(core_axis_name="c", subcore_axis_name="s"),
        scratch_types=_SC_SCRATCH,
    )
    return gk(f0, f1, eidx)


def _gather2(y2, ridx, lidx):
    gk = pl.kernel(
        _gather2_body,
        out_type=jax.ShapeDtypeStruct((32, _E2P), jnp.float32),
        mesh=plsc.VectorSubcoreMesh(core_axis_name="c", subcore_axis_name="s"),
        compiler_params=pltpu.CompilerParams(needs_layout_passes=False),
        scratch_types=[
            pltpu.VMEM((_E2P,), jnp.int32),
            pltpu.VMEM((_E2P,), jnp.int32),
            pltpu.VMEM((_E2P, 52), jnp.float32),
            pltpu.VMEM((_E2P,), jnp.float32),
            pltpu.SemaphoreType.DMA,
        ],
    )
    return gk(y2, ridx, lidx)


def _dense_body(p0_ref, p1_ref, p2_ref, out_ref):
    @pl.when(pl.program_id(0) == 0)
    def _():
        for s in range(3):
            out_ref[s] = 0.0

    for s, ref in enumerate([p0_ref, p1_ref, p2_ref]):
        out_ref[s] += jnp.sum(_bce0(ref[...]))


def _dense(v0, v1, v2):
    specs = [
        pl.BlockSpec((1, 13, 1, 13, 32), lambda a: (a, 0, 4, 0, 0)),
        pl.BlockSpec((1, 26, 1, 32, 26), lambda a: (a, 0, 4, 0, 0)),
        pl.BlockSpec((1, 52, 1, 32, 52), lambda a: (a, 0, 4, 0, 0)),
    ]
    return pl.pallas_call(
        _dense_body,
        grid=(3,),
        in_specs=specs,
        out_specs=pl.BlockSpec(memory_space=pltpu.SMEM),
        out_shape=jax.ShapeDtypeStruct((3,), jnp.float32),
    )(v0, v1, v2)


def _combine_body(gath_ref, tv_ref, win_ref, lab_ref, ds_ref, out_ref):
    total = jnp.float32(0.0)
    lab = lab_ref[...]
    ci = lax.broadcasted_iota(jnp.int32, (_NCLS, _M), 0)
    for s in range(3):
        H = W = _HW[s]
        g = gath_ref[s]
        w = win_ref[s]
        x4 = g[4:5, :]
        s0 = _bce0(x4)
        n_obj = jnp.sum(w)
        s0m = jnp.sum(w * s0)
        sxm = jnp.sum(w * x4)
        sxy = jnp.sum(w * ((g[0:1] - tv_ref[s, 0]) ** 2 + (g[1:2] - tv_ref[s, 1]) ** 2))
        swh = jnp.sum(w * ((jnp.abs(g[2:3]) - tv_ref[s, 2]) ** 2
                           + (jnp.abs(g[3:4]) - tv_ref[s, 3]) ** 2))
        xc = g[5:10, :]
        onehot = ci == lab
        scls = jnp.sum(w * (_bce0(xc) - jnp.where(onehot, xc, 0.0)))
        n_noobj = jnp.float32(_B * _A * H * W) - n_obj
        total = total + (s0m - sxm) / n_obj \
            + 0.5 * (ds_ref[s] - s0m) / n_noobj \
            + 5.0 * (sxy + swh) / (2.0 * n_obj) \
            + scls / (jnp.float32(_NCLS) * n_obj)
    out_ref[0] = total


def _combine(gath, tv, win, lab, dsums):
    return pl.pallas_call(
        _combine_body,
        in_specs=[
            pl.BlockSpec(),
            pl.BlockSpec(),
            pl.BlockSpec(),
            pl.BlockSpec(),
            pl.BlockSpec(memory_space=pltpu.SMEM),
        ],
        out_specs=pl.BlockSpec(memory_space=pltpu.SMEM),
        out_shape=jax.ShapeDtypeStruct((1,), jnp.float32),
    )(gath, tv, win, lab, dsums)


def kernel(pred0, pred1, pred2, boxes, labels):
    v0 = jnp.transpose(pred0, (1, 2, 4, 3, 0))
    v1 = jnp.transpose(pred1, (1, 2, 4, 0, 3))
    v2 = jnp.transpose(pred2, (1, 2, 4, 0, 3))
    boxes_t = jnp.transpose(boxes, (2, 0, 1))
    base, lane, win, tv = _prep(boxes_t)
    ten = jnp.arange(10, dtype=jnp.int32)
    eidx = jnp.stack([
        base[s].reshape(1, _M) + (ten * strd).reshape(10, 1)
        for s, strd in ((0, 13 * 32), (1, _B * 26))
    ]).reshape(2, _CHUNKS, 128)
    gath01 = _gather01(v0.reshape(-1), v1.reshape(-1), eidx)
    pad2 = jnp.zeros((32, _E2P - _E2), jnp.int32)
    r2 = jnp.concatenate([
        (base[2].reshape(1, _M) + (ten * _B).reshape(10, 1)).reshape(32, _E2),
        pad2], axis=1)
    l2 = jnp.concatenate([
        jnp.broadcast_to(lane[2].reshape(1, _M), (10, _M)).reshape(32, _E2),
        pad2], axis=1)
    gath2 = _gather2(v2.reshape(3 * 52 * 10 * _B, 52), r2, l2)
    gath = jnp.concatenate([
        gath01.reshape(2, 10, _M),
        gath2[:, :_E2].reshape(1, 10, _M),
    ])
    dsums = _dense(v0, v1, v2)
    tot = _combine(gath,
                   tv.reshape(3, 4, 1, _M),
                   win.reshape(3, 1, _M),
                   labels.reshape(1, _M).astype(jnp.int32),
                   dsums)
    return tot[0]


# speedup vs baseline: 24.6377x; 1.0095x over previous
import numpy as np
import jax
import jax.numpy as jnp
from jax import lax
from jax.experimental import pallas as pl
from jax.experimental.pallas import tpu as pltpu
from jax.experimental.pallas import tpu_sc as plsc

_ANCHORS = np.array([[[0.02, 0.02], [0.04, 0.02], [0.02, 0.08]],
                     [[0.06, 0.06], [0.10, 0.04], [0.04, 0.14]],
                     [[0.14, 0.14], [0.22, 0.08], [0.10, 0.24]]], dtype=np.float32)
_NCLS = 5
_B, _A, _N = 32, 3, 20
_HW = (13, 26, 52)
_M = _B * _N
_CHUNKS = (10 * _M) // 128


def _bce0(x):
    return jnp.maximum(x, 0.0) + jnp.log1p(jnp.exp(-jnp.abs(x)))


def _prep_body(boxes_ref, base_ref, lane_ref, win_ref, tv_ref):
    bx = boxes_ref[...]
    cx, cy, bw, bh = bx[0], bx[1], bx[2], bx[3]
    b_iota = lax.broadcasted_iota(jnp.int32, (_B, _N), 0)
    n1 = lax.broadcasted_iota(jnp.int32, (_N, _N), 0)
    n2 = lax.broadcasted_iota(jnp.int32, (_N, _N), 1)
    later = n2 > n1
    for s in range(3):
        H = W = _HW[s]
        fW = jnp.float32(W)
        gx, gy, gw, gh = cx * fW, cy * fW, bw * fW, bh * fW
        gi = jnp.clip(gx.astype(jnp.int32), 0, W - 1)
        gj = jnp.clip(gy.astype(jnp.int32), 0, H - 1)
        r = []
        for k in range(3):
            awk = np.float32(_ANCHORS[s, k, 0]) * np.float32(W)
            ahk = np.float32(_ANCHORS[s, k, 1]) * np.float32(H)
            inter = jnp.minimum(gw, awk) * jnp.minimum(gh, ahk)
            union = gw * gh + np.float32(awk * ahk) - inter + 1e-6
            r.append(inter / union)
        best = jnp.where(r[1] > r[0], 1, 0)
        best = jnp.where(r[2] > jnp.maximum(r[0], r[1]), 2, best)
        ahj = best * H + gj
        if s == 0:
            base = ahj * (10 * 13 * 32) + gi * 32 + b_iota
            ln = jnp.zeros_like(base)
        else:
            base = ahj * (10 * _B) + b_iota
            ln = gi
        key = base * 64 + ln
        eq = key[:, :, None] == key[:, None, :]
        dup = jnp.any(eq & later[None], axis=2)
        base_ref[s] = base
        lane_ref[s] = ln
        win_ref[s] = jnp.where(dup, 0.0, 1.0)
        tv_ref[s, 0] = gx - gi.astype(jnp.float32)
        tv_ref[s, 1] = gy - gj.astype(jnp.float32)
        tv_ref[s, 2] = gw
        tv_ref[s, 3] = gh


def _prep(boxes_t):
    return pl.pallas_call(
        _prep_body,
        out_shape=[
            jax.ShapeDtypeStruct((3, _B, _N), jnp.int32),
            jax.ShapeDtypeStruct((3, _B, _N), jnp.int32),
            jax.ShapeDtypeStruct((3, _B, _N), jnp.float32),
            jax.ShapeDtypeStruct((3, 4, _B, _N), jnp.float32),
        ],
    )(boxes_t)


def _gather0_body(f0, eidx, out, idx_v, row_v, sem):
    wid = lax.axis_index("s") * 2 + lax.axis_index("c")
    for j in range(2):
        cid = wid + 32 * j

        @pl.when(cid < _CHUNKS)
        def _():
            pltpu.sync_copy(eidx.at[cid], idx_v)
            pltpu.async_copy(f0.at[idx_v], row_v, sem).wait()
            pltpu.sync_copy(row_v, out.at[cid])


_E2 = 200
_E2P = 208


def _gatherR_body(y1, y2, ridx, lidx, out, ridx_v, lidx_v, rows1_v, rows2_v,
                  out_v, sem):
    wid = lax.axis_index("s") * 2 + lax.axis_index("c")
    for s, (tab, rows_v) in enumerate([(y1, rows1_v), (y2, rows2_v)]):
        pltpu.sync_copy(ridx.at[s, wid], ridx_v)
        pltpu.sync_copy(lidx.at[s, wid], lidx_v)
        cps = []
        for k in range(_E2P // 16):
            rv = ridx_v[pl.ds(16 * k, 16)]
            for t in range(16):
                i = 16 * k + t
                cps.append(pltpu.async_copy(tab.at[rv[t]], rows_v.at[i], sem))
        for cp in cps:
            cp.wait()
        for i in range(_E2P // 16):
            rr = lax.iota(jnp.int32, 16) + 16 * i
            ll = lidx_v[pl.ds(16 * i, 16)]
            out_v[pl.ds(16 * i, 16)] = plsc.load_gather(rows_v, [rr, ll])
        pltpu.sync_copy(out_v, out.at[s, wid])


_SC_SCRATCH = [
    pltpu.VMEM((128,), jnp.int32),
    pltpu.VMEM((128,), jnp.float32),
    pltpu.SemaphoreType.DMA,
]


def _gather0(f0, eidx):
    gk = pl.kernel(
        _gather0_body,
        out_type=jax.ShapeDtypeStruct((_CHUNKS, 128), jnp.float32),
        mesh=plsc.VectorSubcoreMesh(core_axis_name="c", subcore_axis_name="s"),
        scratch_types=_SC_SCRATCH,
    )
    return gk(f0, eidx)


def _gatherR(y1, y2, ridx, lidx):
    gk = pl.kernel(
        _gatherR_body,
        out_type=jax.ShapeDtypeStruct((2, 32, _E2P), jnp.float32),
        mesh=plsc.VectorSubcoreMesh(core_axis_name="c", subcore_axis_name="s"),
        compiler_params=pltpu.CompilerParams(needs_layout_passes=False),
        scratch_types=[
            pltpu.VMEM((_E2P,), jnp.int32),
            pltpu.VMEM((_E2P,), jnp.int32),
            pltpu.VMEM((_E2P, 26), jnp.float32),
            pltpu.VMEM((_E2P, 52), jnp.float32),
            pltpu.VMEM((_E2P,), jnp.float32),
            pltpu.SemaphoreType.DMA,
        ],
    )
    return gk(y1, y2, ridx, lidx)


def _dense_body(p0_ref, p1_ref, p2_ref, out_ref):
    @pl.when(pl.program_id(0) == 0)
    def _():
        for s in range(3):
            out_ref[s] = 0.0

    for s, ref in enumerate([p0_ref, p1_ref, p2_ref]):
        out_ref[s] += jnp.sum(_bce0(ref[...]))


def _dense(v0, v1, v2):
    specs = [
        pl.BlockSpec((1, 13, 1, 13, 32), lambda a: (a, 0, 4, 0, 0)),
        pl.BlockSpec((1, 26, 1, 32, 26), lambda a: (a, 0, 4, 0, 0)),
        pl.BlockSpec((1, 52, 1, 32, 52), lambda a: (a, 0, 4, 0, 0)),
    ]
    return pl.pallas_call(
        _dense_body,
        grid=(3,),
        in_specs=specs,
        out_specs=pl.BlockSpec(memory_space=pltpu.SMEM),
        out_shape=jax.ShapeDtypeStruct((3,), jnp.float32),
    )(v0, v1, v2)


def _combine_body(gath_ref, tv_ref, win_ref, lab_ref, ds_ref, out_ref):
    total = jnp.float32(0.0)
    lab = lab_ref[...]
    ci = lax.broadcasted_iota(jnp.int32, (_NCLS, _M), 0)
    for s in range(3):
        H = W = _HW[s]
        g = gath_ref[s]
        w = win_ref[s]
        x4 = g[4:5, :]
        s0 = _bce0(x4)
        n_obj = jnp.sum(w)
        s0m = jnp.sum(w * s0)
        sxm = jnp.sum(w * x4)
        sxy = jnp.sum(w * ((g[0:1] - tv_ref[s, 0]) ** 2 + (g[1:2] - tv_ref[s, 1]) ** 2))
        swh = jnp.sum(w * ((jnp.abs(g[2:3]) - tv_ref[s, 2]) ** 2
                           + (jnp.abs(g[3:4]) - tv_ref[s, 3]) ** 2))
        xc = g[5:10, :]
        onehot = ci == lab
        scls = jnp.sum(w * (_bce0(xc) - jnp.where(onehot, xc, 0.0)))
        n_noobj = jnp.float32(_B * _A * H * W) - n_obj
        total = total + (s0m - sxm) / n_obj \
            + 0.5 * (ds_ref[s] - s0m) / n_noobj \
            + 5.0 * (sxy + swh) / (2.0 * n_obj) \
            + scls / (jnp.float32(_NCLS) * n_obj)
    out_ref[0] = total


def _combine(gath, tv, win, lab, dsums):
    return pl.pallas_call(
        _combine_body,
        in_specs=[
            pl.BlockSpec(),
            pl.BlockSpec(),
            pl.BlockSpec(),
            pl.BlockSpec(),
            pl.BlockSpec(memory_space=pltpu.SMEM),
        ],
        out_specs=pl.BlockSpec(memory_space=pltpu.SMEM),
        out_shape=jax.ShapeDtypeStruct((1,), jnp.float32),
    )(gath, tv, win, lab, dsums)


def kernel(pred0, pred1, pred2, boxes, labels):
    v0 = jnp.transpose(pred0, (1, 2, 4, 3, 0))
    v1 = jnp.transpose(pred1, (1, 2, 4, 0, 3))
    v2 = jnp.transpose(pred2, (1, 2, 4, 0, 3))
    boxes_t = jnp.transpose(boxes, (2, 0, 1))
    base, lane, win, tv = _prep(boxes_t)
    ten = jnp.arange(10, dtype=jnp.int32)
    eidx = (base[0].reshape(1, _M)
            + (ten * (13 * 32)).reshape(10, 1)).reshape(_CHUNKS, 128)
    gath0 = _gather0(v0.reshape(-1), eidx)
    pad2 = jnp.zeros((32, _E2P - _E2), jnp.int32)
    ridx = jnp.stack([
        jnp.concatenate([
            (base[s].reshape(1, _M) + (ten * _B).reshape(10, 1)).reshape(32, _E2),
            pad2], axis=1)
        for s in (1, 2)
    ])
    lidx = jnp.stack([
        jnp.concatenate([
            jnp.broadcast_to(lane[s].reshape(1, _M), (10, _M)).reshape(32, _E2),
            pad2], axis=1)
        for s in (1, 2)
    ])
    gathr = _gatherR(v1.reshape(3 * 26 * 10 * _B, 26),
                     v2.reshape(3 * 52 * 10 * _B, 52), ridx, lidx)
    gath = jnp.concatenate([
        gath0.reshape(1, 10, _M),
        gathr[:, :, :_E2].reshape(2, 10, _M),
    ])
    dsums = _dense(v0, v1, v2)
    tot = _combine(gath,
                   tv.reshape(3, 4, 1, _M),
                   win.reshape(3, 1, _M),
                   labels.reshape(1, _M).astype(jnp.int32),
                   dsums)
    return tot[0]
